# baseline (speedup 1.0000x reference)
import functools

import jax
import jax.numpy as jnp
from jax import lax
from jax.experimental import pallas as pl
from jax.experimental.pallas import tpu as pltpu
from jax.experimental.pallas import tpu_sc as plsc

N = 10000
E = 320000
D = 128
R = 8

NUM_TILES = 32
EPT = E // NUM_TILES
C = 128
NCH = 80
EPT_PAD = NCH * C
BATCH = 8
NB = NCH // BATCH
PACK = 16384
STRIPE = 640
LAST_STRIPE = N - 15 * STRIPE

BN = 400
GRID = N // BN


def _tc_tables(x_parts, wpack, b, *, fuse_relu_add):
    def body(x_ref, w_ref, b_ref, t_ref, s_ref):
        if fuse_relu_add:
            x = jnp.maximum(x_ref[0] + x_ref[1], 0.0)
        else:
            x = x_ref[...]
        y = jnp.dot(x, w_ref[...], preferred_element_type=jnp.float32)
        for r in range(R):
            t_ref[r] = y[:, r * D:(r + 1) * D]
        s_ref[...] = y[:, R * D:] + b_ref[...]

    if fuse_relu_add:
        x_spec = pl.BlockSpec((2, BN, D), lambda i: (0, i, 0))
    else:
        x_spec = pl.BlockSpec((BN, D), lambda i: (i, 0))
    return pl.pallas_call(
        body,
        grid=(GRID,),
        in_specs=[
            x_spec,
            pl.BlockSpec((D, (R + 1) * D), lambda i: (0, 0)),
            pl.BlockSpec((1, D), lambda i: (0, 0)),
        ],
        out_specs=[
            pl.BlockSpec((R, BN, D), lambda i: (0, i, 0)),
            pl.BlockSpec((BN, D), lambda i: (i, 0)),
        ],
        out_shape=[
            jax.ShapeDtypeStruct((R, N, D), jnp.float32),
            jax.ShapeDtypeStruct((N, D), jnp.float32),
        ],
    )(x_parts, wpack, b)


def _tc_sum2(parts):
    def body(p_ref, o_ref):
        o_ref[...] = p_ref[0] + p_ref[1]

    return pl.pallas_call(
        body,
        grid=(GRID,),
        in_specs=[pl.BlockSpec((2, BN, D), lambda i: (0, i, 0))],
        out_specs=pl.BlockSpec((BN, D), lambda i: (i, 0)),
        out_shape=jax.ShapeDtypeStruct((N, D), jnp.float32),
    )(parts)


_SC_MESH = plsc.VectorSubcoreMesh(core_axis_name="c", subcore_axis_name="s")


@functools.partial(
    pl.kernel,
    out_type=jax.ShapeDtypeStruct((2, N, D), jnp.float32),
    mesh=_SC_MESH,
    scratch_types=[
        pltpu.VMEM((3 * BATCH, C), jnp.int32),
        pltpu.VMEM((3 * BATCH, C), jnp.int32),
        pltpu.VMEM((C, D), jnp.float32),
        pltpu.VMEM((C, D), jnp.float32),
        pltpu.VMEM_SHARED((N + 8, D), jnp.float32),
        pltpu.SemaphoreType.DMA,
        pltpu.SemaphoreType.DMA,
        pltpu.SemaphoreType.DMA,
        pltpu.SemaphoreType.DMA,
        pltpu.SemaphoreType.DMA,
    ],
)
def _sc_aggregate(comb_hbm, dst_hbm, table_hbm, init_hbm, zeros_hbm,
                  out_hbm, fring, dring, rows0, rows1,
                  acc, semd, sg0, sg1, ss0, ss1):
    cid = lax.axis_index("c")
    sid = lax.axis_index("s")
    wid = cid * 16 + sid

    row0 = sid * STRIPE

    def seed(src_hbm_ref):
        @pl.when(sid < 15)
        def _():
            pltpu.sync_copy(src_hbm_ref.at[pl.ds(row0, STRIPE)],
                            acc.at[pl.ds(row0, STRIPE)])

        @pl.when(sid == 15)
        def _():
            pltpu.sync_copy(src_hbm_ref.at[pl.ds(15 * STRIPE, LAST_STRIPE)],
                            acc.at[pl.ds(15 * STRIPE, LAST_STRIPE)])

    @pl.when(cid == 0)
    def _():
        seed(init_hbm)

    @pl.when(cid != 0)
    def _():
        seed(zeros_hbm)

    plsc.subcore_barrier()

    rows_bufs = (rows0, rows1)
    sg = (sg0, sg1)
    ss = (ss0, ss1)

    def issue_batch(b):
        t = (b % 3) * BATCH
        pltpu.async_copy(comb_hbm.at[wid, pl.ds(b * BATCH, BATCH)],
                         fring.at[pl.ds(t, BATCH)], semd)
        pltpu.async_copy(dst_hbm.at[wid, pl.ds(b * BATCH, BATCH)],
                         dring.at[pl.ds(t, BATCH)], semd)

    def drain_batch(b):
        t = (b % 3) * BATCH
        pltpu.make_async_copy(comb_hbm.at[wid, pl.ds(b * BATCH, BATCH)],
                              fring.at[pl.ds(t, BATCH)], semd).wait()
        pltpu.make_async_copy(dst_hbm.at[wid, pl.ds(b * BATCH, BATCH)],
                              dring.at[pl.ds(t, BATCH)], semd).wait()

    issue_batch(0)

    def super_body(k, carry):
        t = (k % 3) * BATCH
        drain_batch(k)

        def flat_body(r, c2):
            for c in range(C // 16):
                sl = pl.ds(c * 16, 16)
                v = fring[t + r, sl]
                fring[t + r, sl] = (
                    lax.shift_right_logical(v, 14) * N
                    + lax.bitwise_and(v, PACK - 1))
            return c2

        lax.fori_loop(0, BATCH, flat_body, 0)

        @pl.when(k < NB - 1)
        def _():
            issue_batch(k + 1)

        for j in range(BATCH):
            pltpu.async_copy(table_hbm.at[fring.at[t + j]], rows_bufs[0],
                             sg[0]).wait()
            pltpu.sync_copy(rows_bufs[0], acc.at[dring.at[t + j]], add=True)
        return carry

    lax.fori_loop(0, NB, super_body, 0)

    plsc.subcore_barrier()

    @pl.when(sid < 15)
    def _():
        pltpu.sync_copy(acc.at[pl.ds(row0, STRIPE)],
                        out_hbm.at[cid, pl.ds(row0, STRIPE)])

    @pl.when(sid == 15)
    def _():
        pltpu.sync_copy(acc.at[pl.ds(15 * STRIPE, LAST_STRIPE)],
                        out_hbm.at[cid, pl.ds(15 * STRIPE, LAST_STRIPE)])


def kernel(feat, edge_index, etypes, W1, Wl1, b1, W2, Wl2, b2):
    pad = EPT_PAD - EPT
    comb = (etypes * PACK + edge_index[0]).reshape(NUM_TILES, EPT)
    comb = jnp.pad(comb, ((0, 0), (0, pad))).reshape(NUM_TILES, NCH, C)
    dst = jnp.pad(edge_index[1].reshape(NUM_TILES, EPT),
                  ((0, 0), (0, pad)),
                  constant_values=N).reshape(NUM_TILES, NCH, C)
    zeros = jnp.zeros((N, D), jnp.float32)

    wpack1 = jnp.concatenate(
        [W1.transpose(1, 0, 2).reshape(D, R * D), Wl1], axis=1)
    wpack2 = jnp.concatenate(
        [W2.transpose(1, 0, 2).reshape(D, R * D), Wl2], axis=1)

    table1, self1 = _tc_tables(feat, wpack1, b1.reshape(1, D),
                               fuse_relu_add=False)
    p1 = _sc_aggregate(comb, dst, table1.reshape(R * N, D), self1, zeros)
    table2, self2 = _tc_tables(p1, wpack2, b2.reshape(1, D),
                               fuse_relu_add=True)
    p2 = _sc_aggregate(comb, dst, table2.reshape(R * N, D), self2, zeros)
    return _tc_sum2(p2)


# speedup vs baseline: 2.7349x; 2.7349x over previous
import functools

import jax
import jax.numpy as jnp
from jax import lax
from jax.experimental import pallas as pl
from jax.experimental.pallas import tpu as pltpu
from jax.experimental.pallas import tpu_sc as plsc

N = 10000
E = 320000
D = 128
R = 8

NUM_TILES = 32
EPT = E // NUM_TILES
C = 80
NCH = EPT // C
BATCH = 8
NB = 16
NCH_PAD = NB * BATCH
PACK = 16384
STRIPE = 640
LAST_STRIPE = N - 15 * STRIPE

BN = 400
GRID = N // BN


def _tc_tables(x_parts, wpack, b, *, fuse_relu_add):
    def body(x_ref, w_ref, b_ref, t_ref, s_ref):
        if fuse_relu_add:
            x = jnp.maximum(x_ref[0] + x_ref[1], 0.0)
        else:
            x = x_ref[...]
        y = jnp.dot(x, w_ref[...], preferred_element_type=jnp.float32)
        for r in range(R):
            t_ref[r] = y[:, r * D:(r + 1) * D]
        s_ref[...] = y[:, R * D:] + b_ref[...]

    if fuse_relu_add:
        x_spec = pl.BlockSpec((2, BN, D), lambda i: (0, i, 0))
    else:
        x_spec = pl.BlockSpec((BN, D), lambda i: (i, 0))
    return pl.pallas_call(
        body,
        grid=(GRID,),
        in_specs=[
            x_spec,
            pl.BlockSpec((D, (R + 1) * D), lambda i: (0, 0)),
            pl.BlockSpec((1, D), lambda i: (0, 0)),
        ],
        out_specs=[
            pl.BlockSpec((R, BN, D), lambda i: (0, i, 0)),
            pl.BlockSpec((BN, D), lambda i: (i, 0)),
        ],
        out_shape=[
            jax.ShapeDtypeStruct((R, N, D), jnp.float32),
            jax.ShapeDtypeStruct((N, D), jnp.float32),
        ],
    )(x_parts, wpack, b)


def _tc_sum2(parts):
    def body(p_ref, o_ref):
        o_ref[...] = p_ref[0] + p_ref[1]

    return pl.pallas_call(
        body,
        grid=(GRID,),
        in_specs=[pl.BlockSpec((2, BN, D), lambda i: (0, i, 0))],
        out_specs=pl.BlockSpec((BN, D), lambda i: (i, 0)),
        out_shape=jax.ShapeDtypeStruct((N, D), jnp.float32),
    )(parts)


_SC_MESH = plsc.VectorSubcoreMesh(core_axis_name="c", subcore_axis_name="s")


@functools.partial(
    pl.kernel,
    out_type=jax.ShapeDtypeStruct((2, N, D), jnp.float32),
    mesh=_SC_MESH,
    scratch_types=[
        pltpu.VMEM((3 * BATCH, C), jnp.int32),
        pltpu.VMEM((3 * BATCH, C), jnp.int32),
        pltpu.VMEM((C, D), jnp.float32),
        pltpu.VMEM((C, D), jnp.float32),
        pltpu.VMEM_SHARED((N, D), jnp.float32),
        pltpu.SemaphoreType.DMA,
        pltpu.SemaphoreType.DMA,
        pltpu.SemaphoreType.DMA,
        pltpu.SemaphoreType.DMA,
        pltpu.SemaphoreType.DMA,
    ],
)
def _sc_aggregate(comb_hbm, dst_hbm, table_hbm, init_hbm, zeros_hbm,
                  out_hbm, fring, dring, rows0, rows1,
                  acc, semd, sg0, sg1, ss0, ss1):
    cid = lax.axis_index("c")
    sid = lax.axis_index("s")
    wid = cid * 16 + sid

    row0 = sid * STRIPE

    def seed(src_hbm_ref):
        @pl.when(sid < 15)
        def _():
            pltpu.sync_copy(src_hbm_ref.at[pl.ds(row0, STRIPE)],
                            acc.at[pl.ds(row0, STRIPE)])

        @pl.when(sid == 15)
        def _():
            pltpu.sync_copy(src_hbm_ref.at[pl.ds(15 * STRIPE, LAST_STRIPE)],
                            acc.at[pl.ds(15 * STRIPE, LAST_STRIPE)])

    @pl.when(cid == 0)
    def _():
        seed(init_hbm)

    @pl.when(cid != 0)
    def _():
        seed(zeros_hbm)

    plsc.subcore_barrier()

    rows_bufs = (rows0, rows1)
    sg = (sg0, sg1)
    ss = (ss0, ss1)

    def issue_batch(b):
        t = (b % 3) * BATCH
        pltpu.async_copy(comb_hbm.at[wid, pl.ds(b * BATCH, BATCH)],
                         fring.at[pl.ds(t, BATCH)], semd)
        pltpu.async_copy(dst_hbm.at[wid, pl.ds(b * BATCH, BATCH)],
                         dring.at[pl.ds(t, BATCH)], semd)

    def drain_batch(b):
        t = (b % 3) * BATCH
        pltpu.make_async_copy(comb_hbm.at[wid, pl.ds(b * BATCH, BATCH)],
                              fring.at[pl.ds(t, BATCH)], semd).wait()
        pltpu.make_async_copy(dst_hbm.at[wid, pl.ds(b * BATCH, BATCH)],
                              dring.at[pl.ds(t, BATCH)], semd).wait()

    def unpack_batch(t):
        def flat_body(r, c2):
            for c in range(C // 16):
                sl = pl.ds(c * 16, 16)
                v = fring[t + r, sl]
                fring[t + r, sl] = (
                    lax.shift_right_logical(v, 14) * N
                    + lax.bitwise_and(v, PACK - 1))
            return c2

        lax.fori_loop(0, BATCH, flat_body, 0)

    def pipe_chunks(t, count):
        dg = [None, None]
        ds = [None, None]
        dg[0] = pltpu.async_copy(table_hbm.at[fring.at[t]], rows_bufs[0],
                                 sg[0])
        if count > 1:
            dg[1] = pltpu.async_copy(table_hbm.at[fring.at[t + 1]],
                                     rows_bufs[1], sg[1])
        for j in range(count):
            p = j % 2
            dg[p].wait()
            ds[p] = pltpu.async_copy(rows_bufs[p], acc.at[dring.at[t + j]],
                                     ss[p], add=True)
            if j + 2 < count:
                ds[p].wait()
                dg[p] = pltpu.async_copy(table_hbm.at[fring.at[t + j + 2]],
                                         rows_bufs[p], sg[p])
        ds[0].wait()
        if count > 1:
            ds[1].wait()

    issue_batch(0)

    def super_body(k, carry):
        t = (k % 3) * BATCH
        drain_batch(k)
        unpack_batch(t)
        issue_batch(k + 1)
        pipe_chunks(t, BATCH)
        return carry

    lax.fori_loop(0, NB - 1, super_body, 0)

    t_last = ((NB - 1) % 3) * BATCH
    drain_batch(NB - 1)
    unpack_batch(t_last)
    pipe_chunks(t_last, NCH - (NB - 1) * BATCH)

    plsc.subcore_barrier()

    @pl.when(sid < 15)
    def _():
        pltpu.sync_copy(acc.at[pl.ds(row0, STRIPE)],
                        out_hbm.at[cid, pl.ds(row0, STRIPE)])

    @pl.when(sid == 15)
    def _():
        pltpu.sync_copy(acc.at[pl.ds(15 * STRIPE, LAST_STRIPE)],
                        out_hbm.at[cid, pl.ds(15 * STRIPE, LAST_STRIPE)])


def kernel(feat, edge_index, etypes, W1, Wl1, b1, W2, Wl2, b2):
    pad = (NCH_PAD - NCH) * C
    comb = (etypes * PACK + edge_index[0]).reshape(NUM_TILES, EPT)
    comb = jnp.pad(comb, ((0, 0), (0, pad))).reshape(NUM_TILES, NCH_PAD, C)
    dst = jnp.pad(edge_index[1].reshape(NUM_TILES, EPT),
                  ((0, 0), (0, pad))).reshape(NUM_TILES, NCH_PAD, C)
    zeros = jnp.zeros((N, D), jnp.float32)

    wpack1 = jnp.concatenate(
        [W1.transpose(1, 0, 2).reshape(D, R * D), Wl1], axis=1)
    wpack2 = jnp.concatenate(
        [W2.transpose(1, 0, 2).reshape(D, R * D), Wl2], axis=1)

    table1, self1 = _tc_tables(feat, wpack1, b1.reshape(1, D),
                               fuse_relu_add=False)
    p1 = _sc_aggregate(comb, dst, table1.reshape(R * N, D), self1, zeros)
    table2, self2 = _tc_tables(p1, wpack2, b2.reshape(1, D),
                               fuse_relu_add=True)
    p2 = _sc_aggregate(comb, dst, table2.reshape(R * N, D), self2, zeros)
    return _tc_sum2(p2)


# speedup vs baseline: 2.9181x; 1.0670x over previous
import functools

import jax
import jax.numpy as jnp
from jax import lax
from jax.experimental import pallas as pl
from jax.experimental.pallas import tpu as pltpu
from jax.experimental.pallas import tpu_sc as plsc

N = 10000
E = 320000
D = 128
R = 8

NUM_TILES = 32
EPT = E // NUM_TILES
C = 80
NCH = EPT // C
BATCH = 8
NB = 16
NCH_PAD = NB * BATCH
PACK = 16384
STRIPE = 640
LAST_STRIPE = N - 15 * STRIPE

BN = 400
GRID = N // BN


def _tc_tables(x_parts, wpack, b, *, fuse_relu_add):
    def body(x_ref, w_ref, b_ref, t_ref, s_ref):
        if fuse_relu_add:
            x = jnp.maximum(x_ref[0] + x_ref[1], 0.0)
        else:
            x = x_ref[...]
        y = jnp.dot(x, w_ref[...], preferred_element_type=jnp.float32)
        for r in range(R):
            t_ref[r] = y[:, r * D:(r + 1) * D]
        s_ref[...] = y[:, R * D:] + b_ref[...]

    if fuse_relu_add:
        x_spec = pl.BlockSpec((2, BN, D), lambda i: (0, i, 0))
    else:
        x_spec = pl.BlockSpec((BN, D), lambda i: (i, 0))
    return pl.pallas_call(
        body,
        grid=(GRID,),
        in_specs=[
            x_spec,
            pl.BlockSpec((D, (R + 1) * D), lambda i: (0, 0)),
            pl.BlockSpec((1, D), lambda i: (0, 0)),
        ],
        out_specs=[
            pl.BlockSpec((R, BN, D), lambda i: (0, i, 0)),
            pl.BlockSpec((BN, D), lambda i: (i, 0)),
        ],
        out_shape=[
            jax.ShapeDtypeStruct((R, N, D), jnp.float32),
            jax.ShapeDtypeStruct((N, D), jnp.float32),
        ],
    )(x_parts, wpack, b)


def _tc_sum2(parts):
    def body(p_ref, o_ref):
        o_ref[...] = p_ref[0] + p_ref[1]

    return pl.pallas_call(
        body,
        grid=(GRID,),
        in_specs=[pl.BlockSpec((2, BN, D), lambda i: (0, i, 0))],
        out_specs=pl.BlockSpec((BN, D), lambda i: (i, 0)),
        out_shape=jax.ShapeDtypeStruct((N, D), jnp.float32),
    )(parts)


_SC_MESH = plsc.VectorSubcoreMesh(core_axis_name="c", subcore_axis_name="s")


@functools.partial(
    pl.kernel,
    out_type=jax.ShapeDtypeStruct((2, N, D), jnp.float32),
    mesh=_SC_MESH,
    scratch_types=[
        pltpu.VMEM((3 * BATCH, C), jnp.int32),
        pltpu.VMEM((3 * BATCH, C), jnp.int32),
        pltpu.VMEM((C, D), jnp.float32),
        pltpu.VMEM((C, D), jnp.float32),
        pltpu.VMEM((C, D), jnp.float32),
        pltpu.VMEM((C, D), jnp.float32),
        pltpu.VMEM_SHARED((N, D), jnp.float32),
        pltpu.SemaphoreType.DMA,
        pltpu.SemaphoreType.DMA,
        pltpu.SemaphoreType.DMA,
        pltpu.SemaphoreType.DMA,
        pltpu.SemaphoreType.DMA,
        pltpu.SemaphoreType.DMA,
        pltpu.SemaphoreType.DMA,
        pltpu.SemaphoreType.DMA,
        pltpu.SemaphoreType.DMA,
    ],
)
def _sc_aggregate(comb_hbm, dst_hbm, table_hbm, init_hbm, zeros_hbm,
                  out_hbm, fring, dring, rows0, rows1, rows2, rows3,
                  acc, semd, sg0, sg1, sg2, sg3, ss0, ss1, ss2, ss3):
    cid = lax.axis_index("c")
    sid = lax.axis_index("s")
    wid = cid * 16 + sid

    row0 = sid * STRIPE

    def seed(src_hbm_ref):
        @pl.when(sid < 15)
        def _():
            pltpu.sync_copy(src_hbm_ref.at[pl.ds(row0, STRIPE)],
                            acc.at[pl.ds(row0, STRIPE)])

        @pl.when(sid == 15)
        def _():
            pltpu.sync_copy(src_hbm_ref.at[pl.ds(15 * STRIPE, LAST_STRIPE)],
                            acc.at[pl.ds(15 * STRIPE, LAST_STRIPE)])

    @pl.when(cid == 0)
    def _():
        seed(init_hbm)

    @pl.when(cid != 0)
    def _():
        seed(zeros_hbm)

    plsc.subcore_barrier()

    rows_bufs = (rows0, rows1, rows2, rows3)
    sg = (sg0, sg1, sg2, sg3)
    ss = (ss0, ss1, ss2, ss3)
    DEPTH = 4

    def issue_batch(b):
        t = (b % 3) * BATCH
        pltpu.async_copy(comb_hbm.at[wid, pl.ds(b * BATCH, BATCH)],
                         fring.at[pl.ds(t, BATCH)], semd)
        pltpu.async_copy(dst_hbm.at[wid, pl.ds(b * BATCH, BATCH)],
                         dring.at[pl.ds(t, BATCH)], semd)

    def drain_batch(b):
        t = (b % 3) * BATCH
        pltpu.make_async_copy(comb_hbm.at[wid, pl.ds(b * BATCH, BATCH)],
                              fring.at[pl.ds(t, BATCH)], semd).wait()
        pltpu.make_async_copy(dst_hbm.at[wid, pl.ds(b * BATCH, BATCH)],
                              dring.at[pl.ds(t, BATCH)], semd).wait()

    def unpack_batch(t):
        def flat_body(r, c2):
            for c in range(C // 16):
                sl = pl.ds(c * 16, 16)
                v = fring[t + r, sl]
                fring[t + r, sl] = (
                    lax.shift_right_logical(v, 14) * N
                    + lax.bitwise_and(v, PACK - 1))
            return c2

        lax.fori_loop(0, BATCH, flat_body, 0)

    def pipe_chunks(t, count):
        dg = [None] * DEPTH
        ds = [None] * DEPTH
        for j in range(min(DEPTH, count)):
            dg[j] = pltpu.async_copy(table_hbm.at[fring.at[t + j]],
                                     rows_bufs[j], sg[j])
        for j in range(count):
            p = j % DEPTH
            dg[p].wait()
            ds[p] = pltpu.async_copy(rows_bufs[p], acc.at[dring.at[t + j]],
                                     ss[p], add=True)
            if j + DEPTH < count:
                ds[p].wait()
                dg[p] = pltpu.async_copy(table_hbm.at[fring.at[t + j + DEPTH]],
                                         rows_bufs[p], sg[p])
        for j in range(max(0, count - DEPTH), count):
            ds[j % DEPTH].wait()

    issue_batch(0)

    def super_body(k, carry):
        t = (k % 3) * BATCH
        drain_batch(k)
        unpack_batch(t)
        issue_batch(k + 1)
        pipe_chunks(t, BATCH)
        return carry

    lax.fori_loop(0, NB - 1, super_body, 0)

    t_last = ((NB - 1) % 3) * BATCH
    drain_batch(NB - 1)
    unpack_batch(t_last)
    pipe_chunks(t_last, NCH - (NB - 1) * BATCH)

    plsc.subcore_barrier()

    @pl.when(sid < 15)
    def _():
        pltpu.sync_copy(acc.at[pl.ds(row0, STRIPE)],
                        out_hbm.at[cid, pl.ds(row0, STRIPE)])

    @pl.when(sid == 15)
    def _():
        pltpu.sync_copy(acc.at[pl.ds(15 * STRIPE, LAST_STRIPE)],
                        out_hbm.at[cid, pl.ds(15 * STRIPE, LAST_STRIPE)])


def kernel(feat, edge_index, etypes, W1, Wl1, b1, W2, Wl2, b2):
    pad = (NCH_PAD - NCH) * C
    comb = (etypes * PACK + edge_index[0]).reshape(NUM_TILES, EPT)
    comb = jnp.pad(comb, ((0, 0), (0, pad))).reshape(NUM_TILES, NCH_PAD, C)
    dst = jnp.pad(edge_index[1].reshape(NUM_TILES, EPT),
                  ((0, 0), (0, pad))).reshape(NUM_TILES, NCH_PAD, C)
    zeros = jnp.zeros((N, D), jnp.float32)

    wpack1 = jnp.concatenate(
        [W1.transpose(1, 0, 2).reshape(D, R * D), Wl1], axis=1)
    wpack2 = jnp.concatenate(
        [W2.transpose(1, 0, 2).reshape(D, R * D), Wl2], axis=1)

    table1, self1 = _tc_tables(feat, wpack1, b1.reshape(1, D),
                               fuse_relu_add=False)
    p1 = _sc_aggregate(comb, dst, table1.reshape(R * N, D), self1, zeros)
    table2, self2 = _tc_tables(p1, wpack2, b2.reshape(1, D),
                               fuse_relu_add=True)
    p2 = _sc_aggregate(comb, dst, table2.reshape(R * N, D), self2, zeros)
    return _tc_sum2(p2)


# speedup vs baseline: 3.1510x; 1.0798x over previous
import functools

import jax
import jax.numpy as jnp
from jax import lax
from jax.experimental import pallas as pl
from jax.experimental.pallas import tpu as pltpu
from jax.experimental.pallas import tpu_sc as plsc

N = 10000
E = 320000
D = 128
R = 8

NUM_TILES = 32
EPT = E // NUM_TILES
C = 80
NCH = EPT // C
BATCH = 16
NB = 8
NCH_PAD = NB * BATCH
NRING = 2
PACK = 16384
STRIPE = 640
LAST_STRIPE = N - 15 * STRIPE

BN = 400
GRID = N // BN


def _tc_tables(x_parts, wpack, b, *, fuse_relu_add):
    def body(x_ref, w_ref, b_ref, t_ref, s_ref):
        if fuse_relu_add:
            x = jnp.maximum(x_ref[0] + x_ref[1], 0.0)
        else:
            x = x_ref[...]
        y = jnp.dot(x, w_ref[...], preferred_element_type=jnp.float32)
        for r in range(R):
            t_ref[r] = y[:, r * D:(r + 1) * D]
        s_ref[...] = y[:, R * D:] + b_ref[...]

    if fuse_relu_add:
        x_spec = pl.BlockSpec((2, BN, D), lambda i: (0, i, 0))
    else:
        x_spec = pl.BlockSpec((BN, D), lambda i: (i, 0))
    return pl.pallas_call(
        body,
        grid=(GRID,),
        in_specs=[
            x_spec,
            pl.BlockSpec((D, (R + 1) * D), lambda i: (0, 0)),
            pl.BlockSpec((1, D), lambda i: (0, 0)),
        ],
        out_specs=[
            pl.BlockSpec((R, BN, D), lambda i: (0, i, 0)),
            pl.BlockSpec((BN, D), lambda i: (i, 0)),
        ],
        out_shape=[
            jax.ShapeDtypeStruct((R, N, D), jnp.float32),
            jax.ShapeDtypeStruct((N, D), jnp.float32),
        ],
    )(x_parts, wpack, b)


def _tc_sum2(parts):
    def body(p_ref, o_ref):
        o_ref[...] = p_ref[0] + p_ref[1]

    return pl.pallas_call(
        body,
        grid=(GRID,),
        in_specs=[pl.BlockSpec((2, BN, D), lambda i: (0, i, 0))],
        out_specs=pl.BlockSpec((BN, D), lambda i: (i, 0)),
        out_shape=jax.ShapeDtypeStruct((N, D), jnp.float32),
    )(parts)


_SC_MESH = plsc.VectorSubcoreMesh(core_axis_name="c", subcore_axis_name="s")


@functools.partial(
    pl.kernel,
    out_type=jax.ShapeDtypeStruct((2, N, D), jnp.float32),
    mesh=_SC_MESH,
    scratch_types=[
        pltpu.VMEM((NRING * BATCH, C), jnp.int32),
        pltpu.VMEM((NRING * BATCH, C), jnp.int32),
        pltpu.VMEM((C, D), jnp.float32),
        pltpu.VMEM((C, D), jnp.float32),
        pltpu.VMEM((C, D), jnp.float32),
        pltpu.VMEM((C, D), jnp.float32),
        pltpu.VMEM_SHARED((N, D), jnp.float32),
        pltpu.SemaphoreType.DMA,
        pltpu.SemaphoreType.DMA,
        pltpu.SemaphoreType.DMA,
        pltpu.SemaphoreType.DMA,
        pltpu.SemaphoreType.DMA,
        pltpu.SemaphoreType.DMA,
        pltpu.SemaphoreType.DMA,
        pltpu.SemaphoreType.DMA,
        pltpu.SemaphoreType.DMA,
    ],
)
def _sc_aggregate(comb_hbm, dst_hbm, table_hbm, init_hbm, zeros_hbm,
                  out_hbm, fring, dring, rows0, rows1, rows2, rows3,
                  acc, semd, sg0, sg1, sg2, sg3, ss0, ss1, ss2, ss3):
    cid = lax.axis_index("c")
    sid = lax.axis_index("s")
    wid = cid * 16 + sid

    row0 = sid * STRIPE

    def seed(src_hbm_ref):
        @pl.when(sid < 15)
        def _():
            pltpu.sync_copy(src_hbm_ref.at[pl.ds(row0, STRIPE)],
                            acc.at[pl.ds(row0, STRIPE)])

        @pl.when(sid == 15)
        def _():
            pltpu.sync_copy(src_hbm_ref.at[pl.ds(15 * STRIPE, LAST_STRIPE)],
                            acc.at[pl.ds(15 * STRIPE, LAST_STRIPE)])

    def do_seed():
        @pl.when(cid == 0)
        def _():
            seed(init_hbm)

        @pl.when(cid != 0)
        def _():
            seed(zeros_hbm)

    rows_bufs = (rows0, rows1, rows2, rows3)
    sg = (sg0, sg1, sg2, sg3)
    ss = (ss0, ss1, ss2, ss3)
    DEPTH = 4

    def issue_batch(b):
        t = (b % NRING) * BATCH
        pltpu.async_copy(comb_hbm.at[wid, pl.ds(b * BATCH, BATCH)],
                         fring.at[pl.ds(t, BATCH)], semd)
        pltpu.async_copy(dst_hbm.at[wid, pl.ds(b * BATCH, BATCH)],
                         dring.at[pl.ds(t, BATCH)], semd)

    def drain_batch(b):
        t = (b % NRING) * BATCH
        pltpu.make_async_copy(comb_hbm.at[wid, pl.ds(b * BATCH, BATCH)],
                              fring.at[pl.ds(t, BATCH)], semd).wait()
        pltpu.make_async_copy(dst_hbm.at[wid, pl.ds(b * BATCH, BATCH)],
                              dring.at[pl.ds(t, BATCH)], semd).wait()

    def unpack_batch(t):
        def flat_body(r, c2):
            for c in range(C // 16):
                sl = pl.ds(c * 16, 16)
                v = fring[t + r, sl]
                fring[t + r, sl] = (
                    lax.shift_right_logical(v, 14) * N
                    + lax.bitwise_and(v, PACK - 1))
            return c2

        lax.fori_loop(0, BATCH, flat_body, 0)

    def pipe_chunks(t, count):
        dg = [None] * DEPTH
        ds = [None] * DEPTH
        for j in range(min(DEPTH, count)):
            dg[j] = pltpu.async_copy(table_hbm.at[fring.at[t + j]],
                                     rows_bufs[j], sg[j])
        for j in range(count):
            p = j % DEPTH
            dg[p].wait()
            ds[p] = pltpu.async_copy(rows_bufs[p], acc.at[dring.at[t + j]],
                                     ss[p], add=True)
            if j + DEPTH < count:
                ds[p].wait()
                dg[p] = pltpu.async_copy(table_hbm.at[fring.at[t + j + DEPTH]],
                                         rows_bufs[p], sg[p])
        for j in range(max(0, count - DEPTH), count):
            ds[j % DEPTH].wait()

    issue_batch(0)
    do_seed()
    plsc.subcore_barrier()

    def super_body(k, carry):
        t = (k % NRING) * BATCH
        drain_batch(k)
        unpack_batch(t)
        issue_batch(k + 1)
        pipe_chunks(t, BATCH)
        return carry

    lax.fori_loop(0, NB - 1, super_body, 0)

    t_last = ((NB - 1) % NRING) * BATCH
    drain_batch(NB - 1)
    unpack_batch(t_last)
    pipe_chunks(t_last, NCH - (NB - 1) * BATCH)

    plsc.subcore_barrier()

    @pl.when(sid < 15)
    def _():
        pltpu.sync_copy(acc.at[pl.ds(row0, STRIPE)],
                        out_hbm.at[cid, pl.ds(row0, STRIPE)])

    @pl.when(sid == 15)
    def _():
        pltpu.sync_copy(acc.at[pl.ds(15 * STRIPE, LAST_STRIPE)],
                        out_hbm.at[cid, pl.ds(15 * STRIPE, LAST_STRIPE)])


def kernel(feat, edge_index, etypes, W1, Wl1, b1, W2, Wl2, b2):
    pad = (NCH_PAD - NCH) * C
    comb = (etypes * PACK + edge_index[0]).reshape(NUM_TILES, EPT)
    comb = jnp.pad(comb, ((0, 0), (0, pad))).reshape(NUM_TILES, NCH_PAD, C)
    dst = jnp.pad(edge_index[1].reshape(NUM_TILES, EPT),
                  ((0, 0), (0, pad))).reshape(NUM_TILES, NCH_PAD, C)
    zeros = jnp.zeros((N, D), jnp.float32)

    wpack1 = jnp.concatenate(
        [W1.transpose(1, 0, 2).reshape(D, R * D), Wl1], axis=1)
    wpack2 = jnp.concatenate(
        [W2.transpose(1, 0, 2).reshape(D, R * D), Wl2], axis=1)

    table1, self1 = _tc_tables(feat, wpack1, b1.reshape(1, D),
                               fuse_relu_add=False)
    p1 = _sc_aggregate(comb, dst, table1.reshape(R * N, D), self1, zeros)
    table2, self2 = _tc_tables(p1, wpack2, b2.reshape(1, D),
                               fuse_relu_add=True)
    p2 = _sc_aggregate(comb, dst, table2.reshape(R * N, D), self2, zeros)
    return _tc_sum2(p2)


# speedup vs baseline: 3.1639x; 1.0041x over previous
import functools

import jax
import jax.numpy as jnp
from jax import lax
from jax.experimental import pallas as pl
from jax.experimental.pallas import tpu as pltpu
from jax.experimental.pallas import tpu_sc as plsc

N = 10000
E = 320000
D = 128
R = 8

NUM_TILES = 32
EPT = E // NUM_TILES
C = 80
NCH = EPT // C
BATCH = 16
NB = 8
LAST_BATCH = NCH - (NB - 1) * BATCH
NRING = 2
PACK = 16384
STRIPE = 640
LAST_STRIPE = N - 15 * STRIPE

BN = 400
GRID = N // BN


def _tc_tables(x_parts, w, wl, b, *, fuse_relu_add):
    def body(x_ref, w_ref, wl_ref, b_ref, t_ref, s_ref):
        if fuse_relu_add:
            x = jnp.maximum(x_ref[0] + x_ref[1], 0.0)
        else:
            x = x_ref[...]
        for r in range(R):
            t_ref[r] = jnp.dot(x, w_ref[r], preferred_element_type=jnp.float32)
        s_ref[...] = (jnp.dot(x, wl_ref[...],
                              preferred_element_type=jnp.float32)
                      + b_ref[...])

    if fuse_relu_add:
        x_spec = pl.BlockSpec((2, BN, D), lambda i: (0, i, 0))
    else:
        x_spec = pl.BlockSpec((BN, D), lambda i: (i, 0))
    return pl.pallas_call(
        body,
        grid=(GRID,),
        in_specs=[
            x_spec,
            pl.BlockSpec((R, D, D), lambda i: (0, 0, 0)),
            pl.BlockSpec((D, D), lambda i: (0, 0)),
            pl.BlockSpec((1, D), lambda i: (0, 0)),
        ],
        out_specs=[
            pl.BlockSpec((R, BN, D), lambda i: (0, i, 0)),
            pl.BlockSpec((BN, D), lambda i: (i, 0)),
        ],
        out_shape=[
            jax.ShapeDtypeStruct((R, N, D), jnp.float32),
            jax.ShapeDtypeStruct((N, D), jnp.float32),
        ],
    )(x_parts, w, wl, b)


def _tc_sum2(parts):
    def body(p_ref, o_ref):
        o_ref[...] = p_ref[0] + p_ref[1]

    return pl.pallas_call(
        body,
        grid=(GRID,),
        in_specs=[pl.BlockSpec((2, BN, D), lambda i: (0, i, 0))],
        out_specs=pl.BlockSpec((BN, D), lambda i: (i, 0)),
        out_shape=jax.ShapeDtypeStruct((N, D), jnp.float32),
    )(parts)


_SC_MESH = plsc.VectorSubcoreMesh(core_axis_name="c", subcore_axis_name="s")


@functools.partial(
    pl.kernel,
    out_type=jax.ShapeDtypeStruct((2, N, D), jnp.float32),
    mesh=_SC_MESH,
    scratch_types=[
        pltpu.VMEM((NRING * BATCH, C), jnp.int32),
        pltpu.VMEM((NRING * BATCH, C), jnp.int32),
        pltpu.VMEM((C, D), jnp.float32),
        pltpu.VMEM((C, D), jnp.float32),
        pltpu.VMEM((C, D), jnp.float32),
        pltpu.VMEM((C, D), jnp.float32),
        pltpu.VMEM_SHARED((N, D), jnp.float32),
        pltpu.SemaphoreType.DMA,
        pltpu.SemaphoreType.DMA,
        pltpu.SemaphoreType.DMA,
        pltpu.SemaphoreType.DMA,
        pltpu.SemaphoreType.DMA,
        pltpu.SemaphoreType.DMA,
        pltpu.SemaphoreType.DMA,
        pltpu.SemaphoreType.DMA,
        pltpu.SemaphoreType.DMA,
    ],
)
def _sc_aggregate(comb_hbm, dst_hbm, table_hbm, init_hbm, zeros_hbm,
                  out_hbm, fring, dring, rows0, rows1, rows2, rows3,
                  acc, semd, sg0, sg1, sg2, sg3, ss0, ss1, ss2, ss3):
    cid = lax.axis_index("c")
    sid = lax.axis_index("s")
    wid = cid * 16 + sid

    row0 = sid * STRIPE

    def seed(src_hbm_ref):
        @pl.when(sid < 15)
        def _():
            pltpu.sync_copy(src_hbm_ref.at[pl.ds(row0, STRIPE)],
                            acc.at[pl.ds(row0, STRIPE)])

        @pl.when(sid == 15)
        def _():
            pltpu.sync_copy(src_hbm_ref.at[pl.ds(15 * STRIPE, LAST_STRIPE)],
                            acc.at[pl.ds(15 * STRIPE, LAST_STRIPE)])

    def do_seed():
        @pl.when(cid == 0)
        def _():
            seed(init_hbm)

        @pl.when(cid != 0)
        def _():
            seed(zeros_hbm)

    rows_bufs = (rows0, rows1, rows2, rows3)
    sg = (sg0, sg1, sg2, sg3)
    ss = (ss0, ss1, ss2, ss3)
    DEPTH = 4

    def issue_batch(b, rows_n=BATCH):
        t = (b % NRING) * BATCH
        pltpu.async_copy(comb_hbm.at[wid, pl.ds(b * BATCH, rows_n)],
                         fring.at[pl.ds(t, rows_n)], semd)
        pltpu.async_copy(dst_hbm.at[wid, pl.ds(b * BATCH, rows_n)],
                         dring.at[pl.ds(t, rows_n)], semd)

    def drain_batch(b, rows_n=BATCH):
        t = (b % NRING) * BATCH
        pltpu.make_async_copy(comb_hbm.at[wid, pl.ds(b * BATCH, rows_n)],
                              fring.at[pl.ds(t, rows_n)], semd).wait()
        pltpu.make_async_copy(dst_hbm.at[wid, pl.ds(b * BATCH, rows_n)],
                              dring.at[pl.ds(t, rows_n)], semd).wait()

    def unpack_batch(t, count=BATCH):
        def flat_body(r, c2):
            for c in range(C // 16):
                sl = pl.ds(c * 16, 16)
                v = fring[t + r, sl]
                fring[t + r, sl] = (
                    lax.shift_right_logical(v, 14) * N
                    + lax.bitwise_and(v, PACK - 1))
            return c2

        lax.fori_loop(0, count, flat_body, 0)

    def pipe_chunks(t, count):
        dg = [None] * DEPTH
        ds = [None] * DEPTH
        for j in range(min(DEPTH, count)):
            dg[j] = pltpu.async_copy(table_hbm.at[fring.at[t + j]],
                                     rows_bufs[j], sg[j])
        for j in range(count):
            p = j % DEPTH
            dg[p].wait()
            ds[p] = pltpu.async_copy(rows_bufs[p], acc.at[dring.at[t + j]],
                                     ss[p], add=True)
            if j + DEPTH < count:
                ds[p].wait()
                dg[p] = pltpu.async_copy(table_hbm.at[fring.at[t + j + DEPTH]],
                                         rows_bufs[p], sg[p])
        for j in range(max(0, count - DEPTH), count):
            ds[j % DEPTH].wait()

    issue_batch(0)
    do_seed()
    plsc.subcore_barrier()

    def super_body(k, carry):
        t = (k % NRING) * BATCH
        drain_batch(k)
        unpack_batch(t)

        @pl.when(k < NB - 2)
        def _():
            issue_batch(k + 1)

        @pl.when(k == NB - 2)
        def _():
            issue_batch(NB - 1, LAST_BATCH)

        pipe_chunks(t, BATCH)
        return carry

    lax.fori_loop(0, NB - 1, super_body, 0)

    t_last = ((NB - 1) % NRING) * BATCH
    drain_batch(NB - 1, LAST_BATCH)
    unpack_batch(t_last, LAST_BATCH)
    pipe_chunks(t_last, LAST_BATCH)

    plsc.subcore_barrier()

    @pl.when(sid < 15)
    def _():
        pltpu.sync_copy(acc.at[pl.ds(row0, STRIPE)],
                        out_hbm.at[cid, pl.ds(row0, STRIPE)])

    @pl.when(sid == 15)
    def _():
        pltpu.sync_copy(acc.at[pl.ds(15 * STRIPE, LAST_STRIPE)],
                        out_hbm.at[cid, pl.ds(15 * STRIPE, LAST_STRIPE)])


def kernel(feat, edge_index, etypes, W1, Wl1, b1, W2, Wl2, b2):
    comb = (etypes * PACK + edge_index[0]).reshape(NUM_TILES, NCH, C)
    dst = edge_index[1].reshape(NUM_TILES, NCH, C)
    zeros = jnp.zeros((N, D), jnp.float32)

    table1, self1 = _tc_tables(feat, W1, Wl1, b1.reshape(1, D),
                               fuse_relu_add=False)
    p1 = _sc_aggregate(comb, dst, table1.reshape(R * N, D), self1, zeros)
    table2, self2 = _tc_tables(p1, W2, Wl2, b2.reshape(1, D),
                               fuse_relu_add=True)
    p2 = _sc_aggregate(comb, dst, table2.reshape(R * N, D), self2, zeros)
    return _tc_sum2(p2)


# speedup vs baseline: 3.1653x; 1.0004x over previous
import functools

import jax
import jax.numpy as jnp
from jax import lax
from jax.experimental import pallas as pl
from jax.experimental.pallas import tpu as pltpu
from jax.experimental.pallas import tpu_sc as plsc

N = 10000
E = 320000
D = 128
R = 8

NUM_TILES = 32
EPT = E // NUM_TILES
C = 80
NCH = EPT // C
BATCH = 16
NB = 8
LAST_BATCH = NCH - (NB - 1) * BATCH
NRING = 2
PACK = 16384
STRIPE = 640
LAST_STRIPE = N - 15 * STRIPE

BN = 400
GRID = N // BN


def _tc_tables(x_parts, w, wl, b, *, fuse_relu_add):
    def body(x_ref, w_ref, wl_ref, b_ref, t_ref, s_ref):
        if fuse_relu_add:
            x = jnp.maximum(x_ref[0] + x_ref[1], 0.0)
        else:
            x = x_ref[...]
        for r in range(R):
            t_ref[r] = jnp.dot(x, w_ref[r], preferred_element_type=jnp.float32)
        s_ref[...] = (jnp.dot(x, wl_ref[...],
                              preferred_element_type=jnp.float32)
                      + b_ref[...])

    if fuse_relu_add:
        x_spec = pl.BlockSpec((2, BN, D), lambda i: (0, i, 0))
    else:
        x_spec = pl.BlockSpec((BN, D), lambda i: (i, 0))
    return pl.pallas_call(
        body,
        grid=(GRID,),
        in_specs=[
            x_spec,
            pl.BlockSpec((R, D, D), lambda i: (0, 0, 0)),
            pl.BlockSpec((D, D), lambda i: (0, 0)),
            pl.BlockSpec((1, D), lambda i: (0, 0)),
        ],
        out_specs=[
            pl.BlockSpec((R, BN, D), lambda i: (0, i, 0)),
            pl.BlockSpec((BN, D), lambda i: (i, 0)),
        ],
        out_shape=[
            jax.ShapeDtypeStruct((R, N, D), jnp.float32),
            jax.ShapeDtypeStruct((N, D), jnp.float32),
        ],
    )(x_parts, w, wl, b)


def _tc_sum2(parts):
    def body(p_ref, o_ref):
        o_ref[...] = p_ref[0] + p_ref[1]

    return pl.pallas_call(
        body,
        grid=(GRID,),
        in_specs=[pl.BlockSpec((2, BN, D), lambda i: (0, i, 0))],
        out_specs=pl.BlockSpec((BN, D), lambda i: (i, 0)),
        out_shape=jax.ShapeDtypeStruct((N, D), jnp.float32),
    )(parts)


_SC_MESH = plsc.VectorSubcoreMesh(core_axis_name="c", subcore_axis_name="s")


@functools.partial(
    pl.kernel,
    out_type=jax.ShapeDtypeStruct((2, N, D), jnp.float32),
    mesh=_SC_MESH,
    scratch_types=[
        pltpu.VMEM((NRING * BATCH, C), jnp.int32),
        pltpu.VMEM((NRING * BATCH, C), jnp.int32),
        pltpu.VMEM((C, D), jnp.float32),
        pltpu.VMEM((C, D), jnp.float32),
        pltpu.VMEM((C, D), jnp.float32),
        pltpu.VMEM((C, D), jnp.float32),
        pltpu.VMEM_SHARED((N, D), jnp.float32),
        pltpu.SemaphoreType.DMA,
        pltpu.SemaphoreType.DMA,
        pltpu.SemaphoreType.DMA,
        pltpu.SemaphoreType.DMA,
        pltpu.SemaphoreType.DMA,
        pltpu.SemaphoreType.DMA,
        pltpu.SemaphoreType.DMA,
        pltpu.SemaphoreType.DMA,
        pltpu.SemaphoreType.DMA,
    ],
)
def _sc_aggregate(comb_hbm, dst_hbm, table_hbm, init_hbm, zeros_hbm,
                  out_hbm, fring, dring, rows0, rows1, rows2, rows3,
                  acc, semd, sg0, sg1, sg2, sg3, ss0, ss1, ss2, ss3):
    cid = lax.axis_index("c")
    sid = lax.axis_index("s")
    wid = cid * 16 + sid

    row0 = sid * STRIPE

    def seed(src_hbm_ref):
        @pl.when(sid < 15)
        def _():
            pltpu.sync_copy(src_hbm_ref.at[pl.ds(row0, STRIPE)],
                            acc.at[pl.ds(row0, STRIPE)])

        @pl.when(sid == 15)
        def _():
            pltpu.sync_copy(src_hbm_ref.at[pl.ds(15 * STRIPE, LAST_STRIPE)],
                            acc.at[pl.ds(15 * STRIPE, LAST_STRIPE)])

    def do_seed():
        @pl.when(cid == 0)
        def _():
            seed(init_hbm)

        @pl.when(cid != 0)
        def _():
            seed(zeros_hbm)

    rows_bufs = (rows0, rows1, rows2, rows3)
    sg = (sg0, sg1, sg2, sg3)
    ss = (ss0, ss1, ss2, ss3)
    DEPTH = 4

    def issue_batch(b, rows_n=BATCH):
        t = (b % NRING) * BATCH
        pltpu.async_copy(comb_hbm.at[wid, pl.ds(b * BATCH, rows_n)],
                         fring.at[pl.ds(t, rows_n)], semd)
        pltpu.async_copy(dst_hbm.at[wid, pl.ds(b * BATCH, rows_n)],
                         dring.at[pl.ds(t, rows_n)], semd)

    def drain_batch(b, rows_n=BATCH):
        t = (b % NRING) * BATCH
        pltpu.make_async_copy(comb_hbm.at[wid, pl.ds(b * BATCH, rows_n)],
                              fring.at[pl.ds(t, rows_n)], semd).wait()
        pltpu.make_async_copy(dst_hbm.at[wid, pl.ds(b * BATCH, rows_n)],
                              dring.at[pl.ds(t, rows_n)], semd).wait()

    def unpack_batch(t, count=BATCH):
        def flat_body(r, c2):
            for c in range(C // 16):
                sl = pl.ds(c * 16, 16)
                v = fring[t + r, sl]
                fring[t + r, sl] = (
                    lax.shift_right_logical(v, 14) * N
                    + lax.bitwise_and(v, PACK - 1))
            return c2

        lax.fori_loop(0, count, flat_body, 0)

    def pipe_chunks(t, count):
        dg = [None] * DEPTH
        ds = [None] * DEPTH
        for j in range(min(DEPTH, count)):
            dg[j] = pltpu.async_copy(table_hbm.at[fring.at[t + j]],
                                     rows_bufs[j], sg[j])
        for j in range(count):
            p = j % DEPTH
            dg[p].wait()
            ds[p] = pltpu.async_copy(rows_bufs[p], acc.at[dring.at[t + j]],
                                     ss[p], add=True)
            if j + DEPTH < count:
                ds[p].wait()
                dg[p] = pltpu.async_copy(table_hbm.at[fring.at[t + j + DEPTH]],
                                         rows_bufs[p], sg[p])
        for j in range(max(0, count - DEPTH), count):
            ds[j % DEPTH].wait()

    issue_batch(0)
    do_seed()
    plsc.subcore_barrier()

    def super_body(k, carry):
        t = (k % NRING) * BATCH
        drain_batch(k)
        unpack_batch(t)

        @pl.when(k < NB - 2)
        def _():
            issue_batch(k + 1)

        @pl.when(k == NB - 2)
        def _():
            issue_batch(NB - 1, LAST_BATCH)

        pipe_chunks(t, BATCH)
        return carry

    lax.fori_loop(0, NB - 1, super_body, 0)

    t_last = ((NB - 1) % NRING) * BATCH
    drain_batch(NB - 1, LAST_BATCH)
    unpack_batch(t_last, LAST_BATCH)
    pipe_chunks(t_last, LAST_BATCH)

    plsc.subcore_barrier()

    @pl.when(sid < 15)
    def _():
        pltpu.sync_copy(acc.at[pl.ds(row0, STRIPE)],
                        out_hbm.at[cid, pl.ds(row0, STRIPE)])

    @pl.when(sid == 15)
    def _():
        pltpu.sync_copy(acc.at[pl.ds(15 * STRIPE, LAST_STRIPE)],
                        out_hbm.at[cid, pl.ds(15 * STRIPE, LAST_STRIPE)])


def kernel(feat, edge_index, etypes, W1, Wl1, b1, W2, Wl2, b2):
    src = edge_index[0:1].reshape(E)
    comb = (etypes * PACK + src).reshape(NUM_TILES, NCH, C)
    dst = edge_index[1:2].reshape(NUM_TILES, NCH, C)
    zeros = jnp.zeros((N, D), jnp.float32)

    table1, self1 = _tc_tables(feat, W1, Wl1, b1.reshape(1, D),
                               fuse_relu_add=False)
    p1 = _sc_aggregate(comb, dst, table1.reshape(R * N, D), self1, zeros)
    table2, self2 = _tc_tables(p1, W2, Wl2, b2.reshape(1, D),
                               fuse_relu_add=True)
    p2 = _sc_aggregate(comb, dst, table2.reshape(R * N, D), self2, zeros)
    return _tc_sum2(p2)
